# Initial kernel scaffold; baseline (speedup 1.0000x reference)
import jax
import jax.numpy as jnp
from jax.experimental import pallas as pl


def kernel(x, edge_index, W1, b1, W2, b2):
    raise NotImplementedError("write your pallas kernel here")


# speedup vs baseline: 13.0581x; 13.0581x over previous
import functools

import jax
import jax.numpy as jnp
from jax import lax
from jax.experimental import pallas as pl
from jax.experimental.pallas import tpu as pltpu
from jax.experimental.pallas import tpu_sc as plsc

N = 10000
E = 160000
C = 256
H = 128
NS = 16
CK = 128
EPT = 10240
CH = EPT // CK
EP = EPT * NS
RPT = 632
NPAD = RPT * NS
RB = 1000

_MESH = dict(core_axis_name="c", subcore_axis_name="s", num_cores=2,
             num_subcores=NS)


def _deg_pallas(dst3):
    @functools.partial(
        pl.kernel,
        out_type=jax.ShapeDtypeStruct((NPAD, 16), jnp.float32),
        mesh=plsc.VectorSubcoreMesh(**_MESH),
        scratch_types=[
            pltpu.VMEM((CH, CK), jnp.int32),
            pltpu.VMEM((CK, 16), jnp.float32),
            pltpu.VMEM((CK, 16), jnp.float32),
            pltpu.VMEM_SHARED((NPAD, 16), jnp.float32),
        ],
    )
    def k(dst_hbm, deg_hbm, dst_v, ones_v, z_v, dacc):
        c = lax.axis_index("c")
        s = lax.axis_index("s")
        one_hot = jnp.where(lax.iota(jnp.int32, 16) == 0,
                            jnp.float32(1.0), jnp.float32(0.0))
        zero = jnp.zeros((16,), jnp.float32)
        base = s * RPT

        @pl.when(c == 0)
        def _():
            @pl.loop(0, CK)
            def _(i):
                ones_v[i, :] = one_hot
                z_v[i, :] = zero

            @pl.loop(0, 4)
            def _(kk):
                pltpu.sync_copy(z_v, dacc.at[pl.ds(base + kk * CK, CK)])

            pltpu.sync_copy(z_v.at[pl.ds(0, RPT - 4 * CK)],
                            dacc.at[pl.ds(base + 4 * CK, RPT - 4 * CK)])
            pltpu.sync_copy(dst_hbm.at[s], dst_v)

        plsc.subcore_barrier()

        @pl.when(c == 0)
        def _():
            @pl.loop(0, CH)
            def _(j):
                pltpu.sync_copy(ones_v, dacc.at[dst_v.at[j]], add=True)

        plsc.subcore_barrier()

        @pl.when(c == 0)
        def _():
            pltpu.sync_copy(dacc.at[pl.ds(base, RPT)],
                            deg_hbm.at[pl.ds(base, RPT)])

    return k(dst3)


def _agg_pallas(hh2, src6, dst5):
    @functools.partial(
        pl.kernel,
        out_type=jax.ShapeDtypeStruct((2 * NPAD, H), jnp.float32),
        mesh=plsc.VectorSubcoreMesh(**_MESH),
        scratch_types=[
            pltpu.VMEM((CH // 2, CK), jnp.int32),
            pltpu.VMEM((CH // 2, CK), jnp.int32),
            pltpu.VMEM((CK, H), jnp.float32),
            pltpu.VMEM((CK, H), jnp.float32),
            pltpu.VMEM_SHARED((NPAD, H), jnp.float32),
            pltpu.SemaphoreType.DMA,
            pltpu.SemaphoreType.DMA,
        ],
    )
    def k(hh_hbm, src_hbm, dst_hbm, zz_hbm, out_hbm, src_v, dst_v, g0, g1, acc,
          s0, s1):
        c = lax.axis_index("c")
        s = lax.axis_index("s")
        base = s * RPT
        half = CH // 2

        pltpu.sync_copy(zz_hbm, acc.at[pl.ds(base, RPT)])
        plsc.subcore_barrier()

        @pl.loop(0, 2)
        def _(p):
            pltpu.sync_copy(src_hbm.at[(c * NS + s) * 2 + p], src_v)
            pltpu.sync_copy(dst_hbm.at[s * 2 + p], dst_v)

            @pl.loop(0, half)
            def _(j):
                pltpu.async_copy(hh_hbm.at[src_v.at[j]], g0, s0).wait()
                pltpu.sync_copy(g0, acc.at[dst_v.at[j]], add=True)

        plsc.subcore_barrier()
        pltpu.sync_copy(acc.at[pl.ds(base, RPT)],
                        out_hbm.at[pl.ds(c * NPAD + base, RPT)])

    return k(hh2, src6, dst5, jnp.zeros((RPT, H), jnp.float32))


def _mm1_body(x_ref, w_ref, o_ref):
    o_ref[...] = jnp.dot(x_ref[...], w_ref[...],
                         preferred_element_type=jnp.float32)


def _mm1(x, W1):
    return pl.pallas_call(
        _mm1_body,
        grid=(N // RB,),
        in_specs=[pl.BlockSpec((RB, C), lambda i: (i, 0)),
                  pl.BlockSpec((C, C), lambda i: (0, 0))],
        out_specs=pl.BlockSpec((RB, C), lambda i: (i, 0)),
        out_shape=jax.ShapeDtypeStruct((N, C), jnp.float32),
    )(x, W1)


def _scale_body(h_ref, deg_ref, o_ref):
    dinv = lax.rsqrt(deg_ref[...][:, :1] + 1.0)
    o_ref[0] = h_ref[...] * dinv


def _scale(h1, deg):
    return pl.pallas_call(
        _scale_body,
        grid=(N // RB, 2),
        in_specs=[pl.BlockSpec((RB, H), lambda i, c: (i, c)),
                  pl.BlockSpec((RB, 16), lambda i, c: (i, 0))],
        out_specs=pl.BlockSpec((1, RB, H), lambda i, c: (c, i, 0)),
        out_shape=jax.ShapeDtypeStruct((2, N, H), jnp.float32),
    )(h1, deg)


def _mid_body(agg_ref, hh_ref, deg_ref, w_ref, b_ref, o_ref):
    t = jnp.concatenate([agg_ref[0] + hh_ref[0], agg_ref[1] + hh_ref[1]],
                        axis=1)
    dinv = lax.rsqrt(deg_ref[...][:, :1] + 1.0)
    z = t * dinv + b_ref[...]
    z = jnp.where(z > 0, z, 0.01 * z)
    h2 = jnp.dot(z, w_ref[...], preferred_element_type=jnp.float32)
    hh2 = h2 * dinv
    o_ref[0] = hh2[:, :H]
    o_ref[1] = hh2[:, H:]


def _mid(agg1, hh1, deg, W2, b1):
    return pl.pallas_call(
        _mid_body,
        grid=(N // RB,),
        in_specs=[pl.BlockSpec((2, RB, H), lambda i: (0, i, 0)),
                  pl.BlockSpec((2, RB, H), lambda i: (0, i, 0)),
                  pl.BlockSpec((RB, 16), lambda i: (i, 0)),
                  pl.BlockSpec((C, C), lambda i: (0, 0)),
                  pl.BlockSpec((1, C), lambda i: (0, 0))],
        out_specs=pl.BlockSpec((2, RB, H), lambda i: (0, i, 0)),
        out_shape=jax.ShapeDtypeStruct((2, N, H), jnp.float32),
    )(agg1, hh1, deg, W2, b1)


def _out_body(agg_ref, hh_ref, deg_ref, b_ref, o_ref):
    t = jnp.concatenate([agg_ref[0] + hh_ref[0], agg_ref[1] + hh_ref[1]],
                        axis=1)
    dinv = lax.rsqrt(deg_ref[...][:, :1] + 1.0)
    o_ref[...] = t * dinv + b_ref[...]


def _combine(agg2, hh2, deg, b2):
    return pl.pallas_call(
        _out_body,
        grid=(N // RB,),
        in_specs=[pl.BlockSpec((2, RB, H), lambda i: (0, i, 0)),
                  pl.BlockSpec((2, RB, H), lambda i: (0, i, 0)),
                  pl.BlockSpec((RB, 16), lambda i: (i, 0)),
                  pl.BlockSpec((1, C), lambda i: (0, 0))],
        out_specs=pl.BlockSpec((RB, C), lambda i: (i, 0)),
        out_shape=jax.ShapeDtypeStruct((N, C), jnp.float32),
    )(agg2, hh2, deg, b2)


_DBG_JAX_AGG = False
_DBG_JAX_DEG = False


def _agg_jax(hh2, srcp, dstp):
    out = jnp.zeros((2, NPAD, H), jnp.float32)
    return out.at[0, dstp].add(hh2[srcp]).at[1, dstp].add(hh2[srcp + N])


def _deg_jax(dstp):
    cnt = jnp.zeros((NPAD,), jnp.float32).at[dstp].add(1.0)
    return jnp.broadcast_to(cnt[:, None], (NPAD, 16))


def kernel(x, edge_index, W1, b1, W2, b2):
    src = edge_index[0].astype(jnp.int32)
    dst = edge_index[1].astype(jnp.int32)
    pad = EP - E
    pi = jnp.arange(pad, dtype=jnp.int32)
    srcp = jnp.concatenate([src, pi % N])
    dstp = jnp.concatenate([dst, N + (pi % 16)])
    src6 = (srcp[None, :] + jnp.array([0, N], jnp.int32)[:, None]
            ).reshape(2 * NS * 2, CH // 2, CK)
    dst5 = dstp.reshape(NS * 2, CH // 2, CK)
    dst3 = dstp.reshape(NS, CH, CK)

    if _DBG_JAX_DEG:
        deg = _deg_jax(dstp)
    else:
        deg = _deg_pallas(dst3)

    h1 = _mm1(x, W1)
    hh1 = _scale(h1, deg)

    if _DBG_JAX_AGG:
        agg1 = _agg_jax(hh1.reshape(2 * N, H), srcp, dstp)
    else:
        agg1 = _agg_pallas(hh1.reshape(2 * N, H), src6, dst5
                           ).reshape(2, NPAD, H)
    hh2 = _mid(agg1, hh1, deg, W2, b1.reshape(1, C))

    if _DBG_JAX_AGG:
        agg2 = _agg_jax(hh2.reshape(2 * N, H), srcp, dstp)
    else:
        agg2 = _agg_pallas(hh2.reshape(2 * N, H), src6, dst5
                           ).reshape(2, NPAD, H)
    return _combine(agg2, hh2, deg, b2.reshape(1, C))


# speedup vs baseline: 18.5937x; 1.4239x over previous
import functools

import jax
import jax.numpy as jnp
from jax import lax
from jax.experimental import pallas as pl
from jax.experimental.pallas import tpu as pltpu
from jax.experimental.pallas import tpu_sc as plsc

N = 10000
E = 160000
C = 256
H = 128
NS = 16
CK = 128
EPT = 10240
CH = EPT // CK
EP = EPT * NS
RPT = 632
NPAD = RPT * NS
RB = 1000

_MESH = dict(core_axis_name="c", subcore_axis_name="s", num_cores=2,
             num_subcores=NS)


def _deg_pallas(dst3):
    @functools.partial(
        pl.kernel,
        out_type=jax.ShapeDtypeStruct((NPAD, 16), jnp.float32),
        mesh=plsc.VectorSubcoreMesh(**_MESH),
        scratch_types=[
            pltpu.VMEM((CH, CK), jnp.int32),
            pltpu.VMEM((CK, 16), jnp.float32),
            pltpu.VMEM((CK, 16), jnp.float32),
            pltpu.VMEM_SHARED((NPAD, 16), jnp.float32),
        ],
    )
    def k(dst_hbm, deg_hbm, dst_v, ones_v, z_v, dacc):
        c = lax.axis_index("c")
        s = lax.axis_index("s")
        one_hot = jnp.where(lax.iota(jnp.int32, 16) == 0,
                            jnp.float32(1.0), jnp.float32(0.0))
        zero = jnp.zeros((16,), jnp.float32)
        base = s * RPT

        @pl.when(c == 0)
        def _():
            @pl.loop(0, CK)
            def _(i):
                ones_v[i, :] = one_hot
                z_v[i, :] = zero

            @pl.loop(0, 4)
            def _(kk):
                pltpu.sync_copy(z_v, dacc.at[pl.ds(base + kk * CK, CK)])

            pltpu.sync_copy(z_v.at[pl.ds(0, RPT - 4 * CK)],
                            dacc.at[pl.ds(base + 4 * CK, RPT - 4 * CK)])
            pltpu.sync_copy(dst_hbm.at[s], dst_v)

        plsc.subcore_barrier()

        @pl.when(c == 0)
        def _():
            @pl.loop(0, CH)
            def _(j):
                pltpu.sync_copy(ones_v, dacc.at[dst_v.at[j]], add=True)

        plsc.subcore_barrier()

        @pl.when(c == 0)
        def _():
            pltpu.sync_copy(dacc.at[pl.ds(base, RPT)],
                            deg_hbm.at[pl.ds(base, RPT)])

    return k(dst3)


def _agg_pallas(hh2, src6, dst5):
    @functools.partial(
        pl.kernel,
        out_type=jax.ShapeDtypeStruct((2 * NPAD, H), jnp.float32),
        mesh=plsc.VectorSubcoreMesh(**_MESH),
        scratch_types=[
            pltpu.VMEM((CH // 2, CK), jnp.int32),
            pltpu.VMEM((CH // 2, CK), jnp.int32),
            pltpu.VMEM((CK, H), jnp.float32),
            pltpu.VMEM((CK, H), jnp.float32),
            pltpu.VMEM_SHARED((NPAD, H), jnp.float32),
            pltpu.SemaphoreType.DMA,
            pltpu.SemaphoreType.DMA,
        ],
    )
    def k(hh_hbm, src_hbm, dst_hbm, zz_hbm, out_hbm, src_v, dst_v, g0, g1, acc,
          s0, s1):
        c = lax.axis_index("c")
        s = lax.axis_index("s")
        base = s * RPT
        half = CH // 2

        pltpu.sync_copy(zz_hbm, acc.at[pl.ds(base, RPT)])
        plsc.subcore_barrier()

        @pl.loop(0, 2)
        def _(p):
            pltpu.sync_copy(src_hbm.at[(c * NS + s) * 2 + p], src_v)
            pltpu.sync_copy(dst_hbm.at[s * 2 + p], dst_v)
            pltpu.async_copy(hh_hbm.at[src_v.at[0]], g0, s0)

            @pl.loop(0, half, step=2)
            def _(j):
                pltpu.async_copy(hh_hbm.at[src_v.at[j + 1]], g1, s1)
                pltpu.make_async_copy(hh_hbm.at[src_v.at[j]], g0, s0).wait()
                pltpu.sync_copy(g0, acc.at[dst_v.at[j]], add=True)

                @pl.when(j + 2 < half)
                def _():
                    pltpu.async_copy(hh_hbm.at[src_v.at[j + 2]], g0, s0)

                pltpu.make_async_copy(hh_hbm.at[src_v.at[j + 1]], g1, s1).wait()
                pltpu.sync_copy(g1, acc.at[dst_v.at[j + 1]], add=True)

        plsc.subcore_barrier()
        pltpu.sync_copy(acc.at[pl.ds(base, RPT)],
                        out_hbm.at[pl.ds(c * NPAD + base, RPT)])

    return k(hh2, src6, dst5, jnp.zeros((RPT, H), jnp.float32))


def _mm1_body(x_ref, w_ref, o_ref):
    o_ref[...] = jnp.dot(x_ref[...], w_ref[...],
                         preferred_element_type=jnp.float32)


def _mm1(x, W1):
    return pl.pallas_call(
        _mm1_body,
        grid=(N // RB,),
        in_specs=[pl.BlockSpec((RB, C), lambda i: (i, 0)),
                  pl.BlockSpec((C, C), lambda i: (0, 0))],
        out_specs=pl.BlockSpec((RB, C), lambda i: (i, 0)),
        out_shape=jax.ShapeDtypeStruct((N, C), jnp.float32),
    )(x, W1)


def _scale_body(h_ref, deg_ref, o_ref):
    dinv = lax.rsqrt(deg_ref[...][:, :1] + 1.0)
    o_ref[0] = h_ref[...] * dinv


def _scale(h1, deg):
    return pl.pallas_call(
        _scale_body,
        grid=(N // RB, 2),
        in_specs=[pl.BlockSpec((RB, H), lambda i, c: (i, c)),
                  pl.BlockSpec((RB, 16), lambda i, c: (i, 0))],
        out_specs=pl.BlockSpec((1, RB, H), lambda i, c: (c, i, 0)),
        out_shape=jax.ShapeDtypeStruct((2, N, H), jnp.float32),
    )(h1, deg)


def _mid_body(agg_ref, hh_ref, deg_ref, w_ref, b_ref, o_ref):
    t = jnp.concatenate([agg_ref[0] + hh_ref[0], agg_ref[1] + hh_ref[1]],
                        axis=1)
    dinv = lax.rsqrt(deg_ref[...][:, :1] + 1.0)
    z = t * dinv + b_ref[...]
    z = jnp.where(z > 0, z, 0.01 * z)
    h2 = jnp.dot(z, w_ref[...], preferred_element_type=jnp.float32)
    hh2 = h2 * dinv
    o_ref[0] = hh2[:, :H]
    o_ref[1] = hh2[:, H:]


def _mid(agg1, hh1, deg, W2, b1):
    return pl.pallas_call(
        _mid_body,
        grid=(N // RB,),
        in_specs=[pl.BlockSpec((2, RB, H), lambda i: (0, i, 0)),
                  pl.BlockSpec((2, RB, H), lambda i: (0, i, 0)),
                  pl.BlockSpec((RB, 16), lambda i: (i, 0)),
                  pl.BlockSpec((C, C), lambda i: (0, 0)),
                  pl.BlockSpec((1, C), lambda i: (0, 0))],
        out_specs=pl.BlockSpec((2, RB, H), lambda i: (0, i, 0)),
        out_shape=jax.ShapeDtypeStruct((2, N, H), jnp.float32),
    )(agg1, hh1, deg, W2, b1)


def _out_body(agg_ref, hh_ref, deg_ref, b_ref, o_ref):
    t = jnp.concatenate([agg_ref[0] + hh_ref[0], agg_ref[1] + hh_ref[1]],
                        axis=1)
    dinv = lax.rsqrt(deg_ref[...][:, :1] + 1.0)
    o_ref[...] = t * dinv + b_ref[...]


def _combine(agg2, hh2, deg, b2):
    return pl.pallas_call(
        _out_body,
        grid=(N // RB,),
        in_specs=[pl.BlockSpec((2, RB, H), lambda i: (0, i, 0)),
                  pl.BlockSpec((2, RB, H), lambda i: (0, i, 0)),
                  pl.BlockSpec((RB, 16), lambda i: (i, 0)),
                  pl.BlockSpec((1, C), lambda i: (0, 0))],
        out_specs=pl.BlockSpec((RB, C), lambda i: (i, 0)),
        out_shape=jax.ShapeDtypeStruct((N, C), jnp.float32),
    )(agg2, hh2, deg, b2)


_DBG_JAX_AGG = False
_DBG_JAX_DEG = False


def _agg_jax(hh2, srcp, dstp):
    out = jnp.zeros((2, NPAD, H), jnp.float32)
    return out.at[0, dstp].add(hh2[srcp]).at[1, dstp].add(hh2[srcp + N])


def _deg_jax(dstp):
    cnt = jnp.zeros((NPAD,), jnp.float32).at[dstp].add(1.0)
    return jnp.broadcast_to(cnt[:, None], (NPAD, 16))


def kernel(x, edge_index, W1, b1, W2, b2):
    src = edge_index[0].astype(jnp.int32)
    dst = edge_index[1].astype(jnp.int32)
    pad = EP - E
    pi = jnp.arange(pad, dtype=jnp.int32)
    srcp = jnp.concatenate([src, pi % N])
    dstp = jnp.concatenate([dst, N + (pi % 16)])
    src6 = (srcp[None, :] + jnp.array([0, N], jnp.int32)[:, None]
            ).reshape(2 * NS * 2, CH // 2, CK)
    dst5 = dstp.reshape(NS * 2, CH // 2, CK)
    dst3 = dstp.reshape(NS, CH, CK)

    if _DBG_JAX_DEG:
        deg = _deg_jax(dstp)
    else:
        deg = _deg_pallas(dst3)

    h1 = _mm1(x, W1)
    hh1 = _scale(h1, deg)

    if _DBG_JAX_AGG:
        agg1 = _agg_jax(hh1.reshape(2 * N, H), srcp, dstp)
    else:
        agg1 = _agg_pallas(hh1.reshape(2 * N, H), src6, dst5
                           ).reshape(2, NPAD, H)
    hh2 = _mid(agg1, hh1, deg, W2, b1.reshape(1, C))

    if _DBG_JAX_AGG:
        agg2 = _agg_jax(hh2.reshape(2 * N, H), srcp, dstp)
    else:
        agg2 = _agg_pallas(hh2.reshape(2 * N, H), src6, dst5
                           ).reshape(2, NPAD, H)
    return _combine(agg2, hh2, deg, b2.reshape(1, C))


# speedup vs baseline: 18.6112x; 1.0009x over previous
import functools

import jax
import jax.numpy as jnp
from jax import lax
from jax.experimental import pallas as pl
from jax.experimental.pallas import tpu as pltpu
from jax.experimental.pallas import tpu_sc as plsc

N = 10000
E = 160000
C = 256
H = 128
NS = 16
CK = 128
EPT = 10240
CH = EPT // CK
EP = EPT * NS
RPT = 632
NPAD = RPT * NS
RB = 1000

_MESH = dict(core_axis_name="c", subcore_axis_name="s", num_cores=2,
             num_subcores=NS)


def _deg_pallas(dst3):
    @functools.partial(
        pl.kernel,
        out_type=jax.ShapeDtypeStruct((2 * NPAD, 16), jnp.float32),
        mesh=plsc.VectorSubcoreMesh(**_MESH),
        scratch_types=[
            pltpu.VMEM((CH // 2, CK), jnp.int32),
            pltpu.VMEM((CK, 16), jnp.float32),
            pltpu.VMEM((CK, 16), jnp.float32),
            pltpu.VMEM_SHARED((NPAD, 16), jnp.float32),
        ],
    )
    def k(dst_hbm, deg_hbm, dst_v, ones_v, z_v, dacc):
        c = lax.axis_index("c")
        s = lax.axis_index("s")
        one_hot = jnp.where(lax.iota(jnp.int32, 16) == 0,
                            jnp.float32(1.0), jnp.float32(0.0))
        zero = jnp.zeros((16,), jnp.float32)
        base = s * RPT

        @pl.loop(0, CK)
        def _(i):
            ones_v[i, :] = one_hot
            z_v[i, :] = zero

        @pl.loop(0, 4)
        def _(kk):
            pltpu.sync_copy(z_v, dacc.at[pl.ds(base + kk * CK, CK)])

        pltpu.sync_copy(z_v.at[pl.ds(0, RPT - 4 * CK)],
                        dacc.at[pl.ds(base + 4 * CK, RPT - 4 * CK)])
        pltpu.sync_copy(dst_hbm.at[s * 2 + c], dst_v)
        plsc.subcore_barrier()

        @pl.loop(0, CH // 2)
        def _(j):
            pltpu.sync_copy(ones_v, dacc.at[dst_v.at[j]], add=True)

        plsc.subcore_barrier()
        pltpu.sync_copy(dacc.at[pl.ds(base, RPT)],
                        deg_hbm.at[pl.ds(c * NPAD + base, RPT)])

    return k(dst3)


def _agg_pallas(hh2, src6, dst5):
    @functools.partial(
        pl.kernel,
        out_type=jax.ShapeDtypeStruct((2 * NPAD, H), jnp.float32),
        mesh=plsc.VectorSubcoreMesh(**_MESH),
        scratch_types=[
            pltpu.VMEM((CH // 2, CK), jnp.int32),
            pltpu.VMEM((CH // 2, CK), jnp.int32),
            pltpu.VMEM((CK, H), jnp.float32),
            pltpu.VMEM((CK, H), jnp.float32),
            pltpu.VMEM_SHARED((NPAD, H), jnp.float32),
            pltpu.SemaphoreType.DMA,
            pltpu.SemaphoreType.DMA,
        ],
    )
    def k(hh_hbm, src_hbm, dst_hbm, zz_hbm, out_hbm, src_v, dst_v, g0, g1, acc,
          s0, s1):
        c = lax.axis_index("c")
        s = lax.axis_index("s")
        base = s * RPT
        half = CH // 2

        pltpu.sync_copy(zz_hbm, acc.at[pl.ds(base, RPT)])
        plsc.subcore_barrier()

        @pl.loop(0, 2)
        def _(p):
            pltpu.sync_copy(src_hbm.at[(c * NS + s) * 2 + p], src_v)
            pltpu.sync_copy(dst_hbm.at[s * 2 + p], dst_v)
            pltpu.async_copy(hh_hbm.at[src_v.at[0]], g0, s0)

            @pl.loop(0, half, step=2)
            def _(j):
                pltpu.async_copy(hh_hbm.at[src_v.at[j + 1]], g1, s1)
                pltpu.make_async_copy(hh_hbm.at[src_v.at[j]], g0, s0).wait()
                pltpu.sync_copy(g0, acc.at[dst_v.at[j]], add=True)

                @pl.when(j + 2 < half)
                def _():
                    pltpu.async_copy(hh_hbm.at[src_v.at[j + 2]], g0, s0)

                pltpu.make_async_copy(hh_hbm.at[src_v.at[j + 1]], g1, s1).wait()
                pltpu.sync_copy(g1, acc.at[dst_v.at[j + 1]], add=True)

        plsc.subcore_barrier()
        pltpu.sync_copy(acc.at[pl.ds(base, RPT)],
                        out_hbm.at[pl.ds(c * NPAD + base, RPT)])

    return k(hh2, src6, dst5, jnp.zeros((RPT, H), jnp.float32))


def _mm1_body(x_ref, w_ref, o_ref):
    o_ref[...] = jnp.dot(x_ref[...], w_ref[...],
                         preferred_element_type=jnp.float32)


def _mm1(x, W1):
    return pl.pallas_call(
        _mm1_body,
        grid=(N // RB,),
        in_specs=[pl.BlockSpec((RB, C), lambda i: (i, 0)),
                  pl.BlockSpec((C, C), lambda i: (0, 0))],
        out_specs=pl.BlockSpec((RB, C), lambda i: (i, 0)),
        out_shape=jax.ShapeDtypeStruct((N, C), jnp.float32),
    )(x, W1)


def _dinv(deg_ref):
    return lax.rsqrt(deg_ref[0][:, :1] + deg_ref[1][:, :1] + 1.0)


def _scale_body(h_ref, deg_ref, o_ref):
    o_ref[0] = h_ref[...] * _dinv(deg_ref)


def _scale(h1, deg):
    return pl.pallas_call(
        _scale_body,
        grid=(N // RB, 2),
        in_specs=[pl.BlockSpec((RB, H), lambda i, c: (i, c)),
                  pl.BlockSpec((2, RB, 16), lambda i, c: (0, i, 0))],
        out_specs=pl.BlockSpec((1, RB, H), lambda i, c: (c, i, 0)),
        out_shape=jax.ShapeDtypeStruct((2, N, H), jnp.float32),
    )(h1, deg)


def _mid_body(agg_ref, hh_ref, deg_ref, w_ref, b_ref, o_ref):
    t = jnp.concatenate([agg_ref[0] + hh_ref[0], agg_ref[1] + hh_ref[1]],
                        axis=1)
    dinv = _dinv(deg_ref)
    z = t * dinv + b_ref[...]
    z = jnp.where(z > 0, z, 0.01 * z)
    h2 = jnp.dot(z, w_ref[...], preferred_element_type=jnp.float32)
    hh2 = h2 * dinv
    o_ref[0] = hh2[:, :H]
    o_ref[1] = hh2[:, H:]


def _mid(agg1, hh1, deg, W2, b1):
    return pl.pallas_call(
        _mid_body,
        grid=(N // RB,),
        in_specs=[pl.BlockSpec((2, RB, H), lambda i: (0, i, 0)),
                  pl.BlockSpec((2, RB, H), lambda i: (0, i, 0)),
                  pl.BlockSpec((2, RB, 16), lambda i: (0, i, 0)),
                  pl.BlockSpec((C, C), lambda i: (0, 0)),
                  pl.BlockSpec((1, C), lambda i: (0, 0))],
        out_specs=pl.BlockSpec((2, RB, H), lambda i: (0, i, 0)),
        out_shape=jax.ShapeDtypeStruct((2, N, H), jnp.float32),
    )(agg1, hh1, deg, W2, b1)


def _out_body(agg_ref, hh_ref, deg_ref, b_ref, o_ref):
    t = jnp.concatenate([agg_ref[0] + hh_ref[0], agg_ref[1] + hh_ref[1]],
                        axis=1)
    o_ref[...] = t * _dinv(deg_ref) + b_ref[...]


def _combine(agg2, hh2, deg, b2):
    return pl.pallas_call(
        _out_body,
        grid=(N // RB,),
        in_specs=[pl.BlockSpec((2, RB, H), lambda i: (0, i, 0)),
                  pl.BlockSpec((2, RB, H), lambda i: (0, i, 0)),
                  pl.BlockSpec((2, RB, 16), lambda i: (0, i, 0)),
                  pl.BlockSpec((1, C), lambda i: (0, 0))],
        out_specs=pl.BlockSpec((RB, C), lambda i: (i, 0)),
        out_shape=jax.ShapeDtypeStruct((N, C), jnp.float32),
    )(agg2, hh2, deg, b2)


_DBG_JAX_AGG = False
_DBG_JAX_DEG = False


def _agg_jax(hh2, srcp, dstp):
    out = jnp.zeros((2, NPAD, H), jnp.float32)
    return out.at[0, dstp].add(hh2[srcp]).at[1, dstp].add(hh2[srcp + N])


def _deg_jax(dstp):
    cnt = jnp.zeros((NPAD,), jnp.float32).at[dstp].add(1.0)
    return jnp.broadcast_to(cnt[:, None], (NPAD, 16))


def kernel(x, edge_index, W1, b1, W2, b2):
    src = edge_index[0].astype(jnp.int32)
    dst = edge_index[1].astype(jnp.int32)
    pad = EP - E
    pi = jnp.arange(pad, dtype=jnp.int32)
    srcp = jnp.concatenate([src, pi % N])
    dstp = jnp.concatenate([dst, N + (pi % 16)])
    src6 = (srcp[None, :] + jnp.array([0, N], jnp.int32)[:, None]
            ).reshape(2 * NS * 2, CH // 2, CK)
    dst5 = dstp.reshape(NS * 2, CH // 2, CK)

    if _DBG_JAX_DEG:
        deg = jnp.stack([_deg_jax(dstp), jnp.zeros((NPAD, 16), jnp.float32)])
    else:
        deg = _deg_pallas(dst5).reshape(2, NPAD, 16)

    h1 = _mm1(x, W1)
    hh1 = _scale(h1, deg)

    if _DBG_JAX_AGG:
        agg1 = _agg_jax(hh1.reshape(2 * N, H), srcp, dstp)
    else:
        agg1 = _agg_pallas(hh1.reshape(2 * N, H), src6, dst5
                           ).reshape(2, NPAD, H)
    hh2 = _mid(agg1, hh1, deg, W2, b1.reshape(1, C))

    if _DBG_JAX_AGG:
        agg2 = _agg_jax(hh2.reshape(2 * N, H), srcp, dstp)
    else:
        agg2 = _agg_pallas(hh2.reshape(2 * N, H), src6, dst5
                           ).reshape(2, NPAD, H)
    return _combine(agg2, hh2, deg, b2.reshape(1, C))


# speedup vs baseline: 18.6234x; 1.0007x over previous
import functools

import jax
import jax.numpy as jnp
from jax import lax
from jax.experimental import pallas as pl
from jax.experimental.pallas import tpu as pltpu
from jax.experimental.pallas import tpu_sc as plsc

N = 10000
E = 160000
C = 256
H = 128
NS = 16
CK = 128
EPT = 10240
CH = EPT // CK
EP = EPT * NS
RPT = 632
NPAD = RPT * NS
RB = 1000

_MESH = dict(core_axis_name="c", subcore_axis_name="s", num_cores=2,
             num_subcores=NS)


def _deg_pallas(dst3):
    @functools.partial(
        pl.kernel,
        out_type=jax.ShapeDtypeStruct((2 * NPAD, 16), jnp.float32),
        mesh=plsc.VectorSubcoreMesh(**_MESH),
        scratch_types=[
            pltpu.VMEM((CH // 2, CK), jnp.int32),
            pltpu.VMEM((CK, 16), jnp.float32),
            pltpu.VMEM((CK, 16), jnp.float32),
            pltpu.VMEM_SHARED((NPAD, 16), jnp.float32),
        ],
    )
    def k(dst_hbm, deg_hbm, dst_v, ones_v, z_v, dacc):
        c = lax.axis_index("c")
        s = lax.axis_index("s")
        one_hot = jnp.where(lax.iota(jnp.int32, 16) == 0,
                            jnp.float32(1.0), jnp.float32(0.0))
        zero = jnp.zeros((16,), jnp.float32)
        base = s * RPT

        @pl.loop(0, CK)
        def _(i):
            ones_v[i, :] = one_hot
            z_v[i, :] = zero

        @pl.loop(0, 4)
        def _(kk):
            pltpu.sync_copy(z_v, dacc.at[pl.ds(base + kk * CK, CK)])

        pltpu.sync_copy(z_v.at[pl.ds(0, RPT - 4 * CK)],
                        dacc.at[pl.ds(base + 4 * CK, RPT - 4 * CK)])
        pltpu.sync_copy(dst_hbm.at[s * 2 + c], dst_v)
        plsc.subcore_barrier()

        @pl.loop(0, CH // 2)
        def _(j):
            pltpu.sync_copy(ones_v, dacc.at[dst_v.at[j]], add=True)

        plsc.subcore_barrier()
        pltpu.sync_copy(dacc.at[pl.ds(base, RPT)],
                        deg_hbm.at[pl.ds(c * NPAD + base, RPT)])

    return k(dst3)


def _agg_pallas(hh2, src6, dst5):
    @functools.partial(
        pl.kernel,
        out_type=jax.ShapeDtypeStruct((2 * NPAD, H), jnp.float32),
        mesh=plsc.VectorSubcoreMesh(**_MESH),
        scratch_types=[
            pltpu.VMEM((CH // 2, CK), jnp.int32),
            pltpu.VMEM((CH // 2, CK), jnp.int32),
            pltpu.VMEM((CK, H), jnp.float32),
            pltpu.VMEM((CK, H), jnp.float32),
            pltpu.VMEM_SHARED((NPAD, H), jnp.float32),
            pltpu.SemaphoreType.DMA,
            pltpu.SemaphoreType.DMA,
        ],
    )
    def k(hh_hbm, src_hbm, dst_hbm, zz_hbm, out_hbm, src_v, dst_v, g0, g1, acc,
          s0, s1):
        c = lax.axis_index("c")
        s = lax.axis_index("s")
        base = s * RPT
        half = CH // 2

        pltpu.sync_copy(zz_hbm, acc.at[pl.ds(base, RPT)])
        plsc.subcore_barrier()

        @pl.loop(0, 2)
        def _(p):
            pltpu.sync_copy(src_hbm.at[(c * NS + s) * 2 + p], src_v)
            pltpu.sync_copy(dst_hbm.at[s * 2 + p], dst_v)
            pltpu.async_copy(hh_hbm.at[src_v.at[0]], g0, s0)

            @pl.loop(0, half, step=2)
            def _(j):
                pltpu.async_copy(hh_hbm.at[src_v.at[j + 1]], g1, s1)
                pltpu.make_async_copy(hh_hbm.at[src_v.at[j]], g0, s0).wait()
                pltpu.sync_copy(g0, acc.at[dst_v.at[j]], add=True)

                @pl.when(j + 2 < half)
                def _():
                    pltpu.async_copy(hh_hbm.at[src_v.at[j + 2]], g0, s0)

                pltpu.make_async_copy(hh_hbm.at[src_v.at[j + 1]], g1, s1).wait()
                pltpu.sync_copy(g1, acc.at[dst_v.at[j + 1]], add=True)

        plsc.subcore_barrier()
        pltpu.sync_copy(acc.at[pl.ds(base, RPT)],
                        out_hbm.at[pl.ds(c * NPAD + base, RPT)])

    return k(hh2, src6, dst5, jnp.zeros((RPT, H), jnp.float32))


def _mm1_body(x_ref, w_ref, o_ref):
    o_ref[...] = jnp.dot(x_ref[...], w_ref[...],
                         preferred_element_type=jnp.float32)


def _mm1(x, W1):
    return pl.pallas_call(
        _mm1_body,
        grid=(N // RB,),
        in_specs=[pl.BlockSpec((RB, C), lambda i: (i, 0)),
                  pl.BlockSpec((C, C), lambda i: (0, 0))],
        out_specs=pl.BlockSpec((RB, C), lambda i: (i, 0)),
        out_shape=jax.ShapeDtypeStruct((N, C), jnp.float32),
    )(x, W1)


def _dinv(deg_ref):
    return lax.rsqrt(deg_ref[0][:, :1] + deg_ref[1][:, :1] + 1.0)


def _scale_body(h_ref, deg_ref, o_ref):
    o_ref[0] = h_ref[...] * _dinv(deg_ref)


def _scale(h1, deg):
    return pl.pallas_call(
        _scale_body,
        grid=(N // RB, 2),
        in_specs=[pl.BlockSpec((RB, H), lambda i, c: (i, c)),
                  pl.BlockSpec((2, RB, 16), lambda i, c: (0, i, 0))],
        out_specs=pl.BlockSpec((1, RB, H), lambda i, c: (c, i, 0)),
        out_shape=jax.ShapeDtypeStruct((2, N, H), jnp.float32),
    )(h1, deg)


def _mid_body(agg_ref, hh_ref, deg_ref, w_ref, b_ref, o_ref):
    t = jnp.concatenate([agg_ref[0] + hh_ref[0], agg_ref[1] + hh_ref[1]],
                        axis=1)
    dinv = _dinv(deg_ref)
    z = t * dinv + b_ref[...]
    z = jnp.where(z > 0, z, 0.01 * z)
    h2 = jnp.dot(z, w_ref[...], preferred_element_type=jnp.float32)
    hh2 = h2 * dinv
    o_ref[0] = hh2[:, :H]
    o_ref[1] = hh2[:, H:]


def _mid(agg1, hh1, deg, W2, b1):
    return pl.pallas_call(
        _mid_body,
        grid=(N // RB,),
        in_specs=[pl.BlockSpec((2, RB, H), lambda i: (0, i, 0)),
                  pl.BlockSpec((2, RB, H), lambda i: (0, i, 0)),
                  pl.BlockSpec((2, RB, 16), lambda i: (0, i, 0)),
                  pl.BlockSpec((C, C), lambda i: (0, 0)),
                  pl.BlockSpec((1, C), lambda i: (0, 0))],
        out_specs=pl.BlockSpec((2, RB, H), lambda i: (0, i, 0)),
        out_shape=jax.ShapeDtypeStruct((2, N, H), jnp.float32),
    )(agg1, hh1, deg, W2, b1)


def _out_body(agg_ref, hh_ref, deg_ref, b_ref, o_ref):
    t = jnp.concatenate([agg_ref[0] + hh_ref[0], agg_ref[1] + hh_ref[1]],
                        axis=1)
    o_ref[...] = t * _dinv(deg_ref) + b_ref[...]


def _combine(agg2, hh2, deg, b2):
    return pl.pallas_call(
        _out_body,
        grid=(N // RB,),
        in_specs=[pl.BlockSpec((2, RB, H), lambda i: (0, i, 0)),
                  pl.BlockSpec((2, RB, H), lambda i: (0, i, 0)),
                  pl.BlockSpec((2, RB, 16), lambda i: (0, i, 0)),
                  pl.BlockSpec((1, C), lambda i: (0, 0))],
        out_specs=pl.BlockSpec((RB, C), lambda i: (i, 0)),
        out_shape=jax.ShapeDtypeStruct((N, C), jnp.float32),
    )(agg2, hh2, deg, b2)


def kernel(x, edge_index, W1, b1, W2, b2):
    src = edge_index[0].astype(jnp.int32)
    dst = edge_index[1].astype(jnp.int32)
    pad = EP - E
    pi = jnp.arange(pad, dtype=jnp.int32)
    srcp = jnp.concatenate([src, pi % N])
    dstp = jnp.concatenate([dst, N + (pi % 16)])
    src6 = (srcp[None, :] + jnp.array([0, N], jnp.int32)[:, None]
            ).reshape(2 * NS * 2, CH // 2, CK)
    dst5 = dstp.reshape(NS * 2, CH // 2, CK)

    deg = _deg_pallas(dst5).reshape(2, NPAD, 16)

    h1 = _mm1(x, W1)
    hh1 = _scale(h1, deg)

    agg1 = _agg_pallas(hh1.reshape(2 * N, H), src6, dst5
                       ).reshape(2, NPAD, H)
    hh2 = _mid(agg1, hh1, deg, W2, b1.reshape(1, C))

    agg2 = _agg_pallas(hh2.reshape(2 * N, H), src6, dst5
                       ).reshape(2, NPAD, H)
    return _combine(agg2, hh2, deg, b2.reshape(1, C))
